# Initial kernel scaffold; baseline (speedup 1.0000x reference)
import jax
import jax.numpy as jnp
from jax.experimental import pallas as pl


def kernel(x, embed):
    raise NotImplementedError("write your pallas kernel here")


# speedup vs baseline: 1.6735x; 1.6735x over previous
import jax
import jax.numpy as jnp
from jax.experimental import pallas as pl

BATCH = 4
SEQ_LEN = 4096
DIM = 2048
BS = 512


def _add_body(x_ref, e_ref, o_ref):
    o_ref[...] = x_ref[...] + e_ref[...][None]


def kernel(x, embed):
    grid = (SEQ_LEN // BS, BATCH)
    return pl.pallas_call(
        _add_body,
        grid=grid,
        in_specs=[
            pl.BlockSpec((1, BS, DIM), lambda s, b: (b, s, 0)),
            pl.BlockSpec((BS, DIM), lambda s, b: (s, 0)),
        ],
        out_specs=pl.BlockSpec((1, BS, DIM), lambda s, b: (b, s, 0)),
        out_shape=jax.ShapeDtypeStruct((BATCH, SEQ_LEN, DIM), jnp.float32),
    )(x, embed)
